# Initial kernel scaffold; baseline (speedup 1.0000x reference)
import jax
import jax.numpy as jnp
from jax.experimental import pallas as pl


def kernel(x, w_gate_up, w_down, retain_gate, retain_up, retain_down, forget_gate, forget_up, forget_down, scales, token_experiment_ids):
    raise NotImplementedError("write your pallas kernel here")


# speedup vs baseline: 1.5766x; 1.5766x over previous
import functools

import jax
import jax.numpy as jnp
from jax.experimental import pallas as pl
from jax.experimental.pallas import tpu as pltpu

T = 2048
H = 1024
FF = 4096
A = 8
NR = 128
NF = 128

NBLK = 8
FFB = FF // NBLK
ADB = 2 * A * NR // NBLK


def _fused_body(x_ref, wgu_ref, wd_ref, wag_ref, wau_ref, wad_ref, ids_ref,
                out_ref, acc_ref):
    j = pl.program_id(0)

    xb = x_ref[...]

    wg = wgu_ref[:, 0, :].astype(jnp.bfloat16)
    wu = wgu_ref[:, 1, :].astype(jnp.bfloat16)
    g = jnp.dot(xb, wg, preferred_element_type=jnp.float32)
    u = jnp.dot(xb, wu, preferred_element_type=jnp.float32)
    hmid = (g * jax.nn.sigmoid(g) * u).astype(jnp.bfloat16)
    wd = wd_ref[...].astype(jnp.bfloat16)
    partial = jnp.dot(hmid, wd, preferred_element_type=jnp.float32)

    ag = jnp.dot(xb, wag_ref[...], preferred_element_type=jnp.float32)
    au = jnp.dot(xb, wau_ref[...], preferred_element_type=jnp.float32)
    inter = ag * jax.nn.sigmoid(ag) * au

    col = jax.lax.broadcasted_iota(jnp.int32, (T, ADB), 1) + j * ADB
    aid = (col % (A * NR)) // NR
    keep = aid == ids_ref[...]
    masked = jnp.where(keep, inter, 0.0).astype(jnp.bfloat16)
    partial = partial + jnp.dot(masked, wad_ref[...],
                                preferred_element_type=jnp.float32)

    @pl.when(j == 0)
    def _init():
        acc_ref[...] = partial

    @pl.when(j > 0)
    def _acc():
        acc_ref[...] += partial

    @pl.when(j == pl.num_programs(0) - 1)
    def _fin():
        out_ref[...] = acc_ref[...]


@jax.jit
def _run(x, w_gate_up, w_down, retain_gate, retain_up, retain_down,
         forget_gate, forget_up, forget_down, scales, token_experiment_ids):
    xb = x.astype(jnp.bfloat16)
    wgu3 = w_gate_up.reshape(H, 2, FF)

    w_ag = jnp.concatenate([retain_gate.reshape(A * NR, H),
                            forget_gate.reshape(A * NF, H)], axis=0)
    w_au = jnp.concatenate([retain_up.reshape(A * NR, H),
                            forget_up.reshape(A * NF, H)], axis=0)
    w_ag = w_ag.T.astype(jnp.bfloat16)
    w_au = w_au.T.astype(jnp.bfloat16)
    rd = (retain_down * scales[:, 0][:, None, None]).transpose(0, 2, 1)
    fd = (forget_down * scales[:, 1][:, None, None]).transpose(0, 2, 1)
    w_ad = jnp.concatenate([rd.reshape(A * NR, H),
                            fd.reshape(A * NF, H)], axis=0).astype(jnp.bfloat16)

    ids = token_experiment_ids.astype(jnp.int32).reshape(T, 1)

    grid = (NBLK,)
    out = pl.pallas_call(
        _fused_body,
        grid=grid,
        in_specs=[
            pl.BlockSpec((T, H), lambda j: (0, 0)),
            pl.BlockSpec((H, 2, FFB), lambda j: (0, 0, j)),
            pl.BlockSpec((FFB, H), lambda j: (j, 0)),
            pl.BlockSpec((H, ADB), lambda j: (0, j)),
            pl.BlockSpec((H, ADB), lambda j: (0, j)),
            pl.BlockSpec((ADB, H), lambda j: (j, 0)),
            pl.BlockSpec((T, 1), lambda j: (0, 0)),
        ],
        out_specs=pl.BlockSpec((T, H), lambda j: (0, 0)),
        out_shape=jax.ShapeDtypeStruct((T, H), jnp.float32),
        scratch_shapes=[pltpu.VMEM((T, H), jnp.float32)],
        compiler_params=pltpu.CompilerParams(
            dimension_semantics=("arbitrary",),
        ),
    )(xb, wgu3, w_down, w_ag, w_au, w_ad, ids)
    return out


def kernel(x, w_gate_up, w_down, retain_gate, retain_up, retain_down,
           forget_gate, forget_up, forget_down, scales, token_experiment_ids):
    return _run(x, w_gate_up, w_down, retain_gate, retain_up, retain_down,
                forget_gate, forget_up, forget_down, scales,
                token_experiment_ids)


# speedup vs baseline: 1.8806x; 1.1928x over previous
import jax
import jax.numpy as jnp
from jax import lax
from jax.experimental import pallas as pl
from jax.experimental.pallas import tpu as pltpu

T = 2048
H = 1024
FF = 4096
A = 8
NR = 128
NF = 128

NBLK = 8
FFB = FF // NBLK
ANB = 2 * NR


def _nt_dot(a, b):
    return lax.dot_general(a, b, (((1,), (1,)), ((), ())),
                           preferred_element_type=jnp.float32)


def _fused_body(x_ref, wgu_ref, wd_ref, rg_ref, ru_ref, rd_ref,
                fg_ref, fu_ref, fd_ref, ids_ref, scales_ref,
                out_ref):
    j = pl.program_id(0)
    xb = x_ref[...]

    wg = wgu_ref[:, 0, :].astype(jnp.bfloat16)
    wu = wgu_ref[:, 1, :].astype(jnp.bfloat16)
    g = jnp.dot(xb, wg, preferred_element_type=jnp.float32)
    u = jnp.dot(xb, wu, preferred_element_type=jnp.float32)
    hmid = (g * jax.nn.sigmoid(g) * u).astype(jnp.bfloat16)
    wd = wd_ref[...].astype(jnp.bfloat16)
    base = jnp.dot(hmid, wd, preferred_element_type=jnp.float32)

    ids = ids_ref[...]
    p = j % (NBLK // 2)
    a0 = 2 * p

    def adapter_partial(g_ref, u_ref, d_ref, branch):
        ag = _nt_dot(xb, g_ref[...].astype(jnp.bfloat16))
        au = _nt_dot(xb, u_ref[...].astype(jnp.bfloat16))
        inter = ag * jax.nn.sigmoid(ag) * au
        sv0 = jnp.where(ids == a0, scales_ref[a0, branch], 0.0)
        sv1 = jnp.where(ids == a0 + 1, scales_ref[a0 + 1, branch], 0.0)
        masked = jnp.concatenate(
            [inter[:, :NR] * sv0, inter[:, NR:] * sv1], axis=1
        ).astype(jnp.bfloat16)
        dw = jnp.concatenate([d_ref[0], d_ref[1]],
                             axis=1).astype(jnp.bfloat16)
        return _nt_dot(masked, dw)

    @pl.when(j < NBLK // 2)
    def _retain():
        adp = adapter_partial(rg_ref, ru_ref, rd_ref, 0)

        @pl.when(j == 0)
        def _init():
            out_ref[...] = base + adp

        @pl.when(j > 0)
        def _acc():
            out_ref[...] += base + adp

    @pl.when(j >= NBLK // 2)
    def _forget():
        out_ref[...] += base + adapter_partial(fg_ref, fu_ref, fd_ref, 1)


@jax.jit
def _run(x, w_gate_up, w_down, retain_gate, retain_up, retain_down,
         forget_gate, forget_up, forget_down, scales, token_experiment_ids):
    wgu3 = w_gate_up.reshape(H, 2, FF)
    rg = retain_gate.reshape(A * NR, H)
    ru = retain_up.reshape(A * NR, H)
    fg = forget_gate.reshape(A * NF, H)
    fu = forget_up.reshape(A * NF, H)
    ids = token_experiment_ids.astype(jnp.int32).reshape(T, 1)
    xb = x.astype(jnp.bfloat16)

    def r_idx(j):
        return (jnp.where(j < 4, j, 0), 0)

    def f_idx(j):
        return (jnp.where(j >= 4, j - 4, 0), 0)

    def rd_idx(j):
        return (jnp.where(j < 4, j, 0), 0, 0)

    def fd_idx(j):
        return (jnp.where(j >= 4, j - 4, 0), 0, 0)

    out = pl.pallas_call(
        _fused_body,
        grid=(NBLK,),
        in_specs=[
            pl.BlockSpec((T, H), lambda j: (0, 0)),
            pl.BlockSpec((H, 2, FFB), lambda j: (0, 0, j)),
            pl.BlockSpec((FFB, H), lambda j: (j, 0)),
            pl.BlockSpec((ANB, H), r_idx),
            pl.BlockSpec((ANB, H), r_idx),
            pl.BlockSpec((2, H, NR), rd_idx),
            pl.BlockSpec((ANB, H), f_idx),
            pl.BlockSpec((ANB, H), f_idx),
            pl.BlockSpec((2, H, NF), fd_idx),
            pl.BlockSpec((T, 1), lambda j: (0, 0)),
            pl.BlockSpec(memory_space=pltpu.SMEM),
        ],
        out_specs=pl.BlockSpec((T, H), lambda j: (0, 0)),
        out_shape=jax.ShapeDtypeStruct((T, H), jnp.float32),
        compiler_params=pltpu.CompilerParams(
            dimension_semantics=("arbitrary",),
            vmem_limit_bytes=63 * 1024 * 1024,
        ),
    )(xb, wgu3, w_down, rg, ru, retain_down, fg, fu, forget_down, ids, scales)
    return out


def kernel(x, w_gate_up, w_down, retain_gate, retain_up, retain_down,
           forget_gate, forget_up, forget_down, scales, token_experiment_ids):
    return _run(x, w_gate_up, w_down, retain_gate, retain_up, retain_down,
                forget_gate, forget_up, forget_down, scales,
                token_experiment_ids)


# speedup vs baseline: 2.8615x; 1.5216x over previous
import jax
import jax.numpy as jnp
from jax import lax
from jax.experimental import pallas as pl
from jax.experimental.pallas import tpu as pltpu

T = 2048
H = 1024
FF = 4096
A = 8
NR = 128
NF = 128

NBLK = 8
FFB = FF // NBLK
ANB = 2 * NR


def _nt_dot(a, b):
    return lax.dot_general(a, b, (((1,), (1,)), ((), ())),
                           preferred_element_type=jnp.float32)


def _fused_body(x_ref, wg_ref, wu_ref, wd_ref, rg_ref, ru_ref, rd_ref,
                fg_ref, fu_ref, fd_ref, ids_ref, scales_ref,
                out_ref):
    j = pl.program_id(0)
    xb = x_ref[...]

    wg = wg_ref[...].astype(jnp.bfloat16)
    wu = wu_ref[...].astype(jnp.bfloat16)
    g = jnp.dot(xb, wg, preferred_element_type=jnp.float32)
    u = jnp.dot(xb, wu, preferred_element_type=jnp.float32)
    hmid = (g * jax.nn.sigmoid(g) * u).astype(jnp.bfloat16)
    wd = wd_ref[...].astype(jnp.bfloat16)
    base = jnp.dot(hmid, wd, preferred_element_type=jnp.float32)

    ids = ids_ref[...]
    p = j % (NBLK // 2)
    a0 = 2 * p

    def adapter_partial(g_ref, u_ref, d_ref, branch):
        ag = _nt_dot(xb, g_ref[...].astype(jnp.bfloat16))
        au = _nt_dot(xb, u_ref[...].astype(jnp.bfloat16))
        inter = ag * jax.nn.sigmoid(ag) * au
        sv0 = jnp.where(ids == a0, scales_ref[a0, branch], 0.0)
        sv1 = jnp.where(ids == a0 + 1, scales_ref[a0 + 1, branch], 0.0)
        masked = jnp.concatenate(
            [inter[:, :NR] * sv0, inter[:, NR:] * sv1], axis=1
        ).astype(jnp.bfloat16)
        dw = jnp.concatenate([d_ref[0], d_ref[1]],
                             axis=1).astype(jnp.bfloat16)
        return _nt_dot(masked, dw)

    @pl.when(j < NBLK // 2)
    def _retain():
        adp = adapter_partial(rg_ref, ru_ref, rd_ref, 0)

        @pl.when(j == 0)
        def _init():
            out_ref[...] = base + adp

        @pl.when(j > 0)
        def _acc():
            out_ref[...] += base + adp

    @pl.when(j >= NBLK // 2)
    def _forget():
        out_ref[...] += base + adapter_partial(fg_ref, fu_ref, fd_ref, 1)


@jax.jit
def _run(x, w_gate_up, w_down, retain_gate, retain_up, retain_down,
         forget_gate, forget_up, forget_down, scales, token_experiment_ids):
    rg = retain_gate.reshape(A * NR, H)
    ru = retain_up.reshape(A * NR, H)
    fg = forget_gate.reshape(A * NF, H)
    fu = forget_up.reshape(A * NF, H)
    ids = token_experiment_ids.astype(jnp.int32).reshape(T, 1)
    xb = x.astype(jnp.bfloat16)

    def r_idx(j):
        return (jnp.where(j < 4, j, 0), 0)

    def f_idx(j):
        return (jnp.where(j >= 4, j - 4, 0), 0)

    def rd_idx(j):
        return (jnp.where(j < 4, j, 0), 0, 0)

    def fd_idx(j):
        return (jnp.where(j >= 4, j - 4, 0), 0, 0)

    out = pl.pallas_call(
        _fused_body,
        grid=(NBLK,),
        in_specs=[
            pl.BlockSpec((T, H), lambda j: (0, 0)),
            pl.BlockSpec((H, FFB), lambda j: (0, j)),
            pl.BlockSpec((H, FFB), lambda j: (0, j + NBLK)),
            pl.BlockSpec((FFB, H), lambda j: (j, 0)),
            pl.BlockSpec((ANB, H), r_idx),
            pl.BlockSpec((ANB, H), r_idx),
            pl.BlockSpec((2, H, NR), rd_idx),
            pl.BlockSpec((ANB, H), f_idx),
            pl.BlockSpec((ANB, H), f_idx),
            pl.BlockSpec((2, H, NF), fd_idx),
            pl.BlockSpec((T, 1), lambda j: (0, 0)),
            pl.BlockSpec(memory_space=pltpu.SMEM),
        ],
        out_specs=pl.BlockSpec((T, H), lambda j: (0, 0)),
        out_shape=jax.ShapeDtypeStruct((T, H), jnp.float32),
        compiler_params=pltpu.CompilerParams(
            dimension_semantics=("arbitrary",),
            vmem_limit_bytes=63 * 1024 * 1024,
        ),
    )(xb, w_gate_up, w_gate_up, w_down, rg, ru, retain_down, fg, fu,
      forget_down, ids, scales)
    return out


def kernel(x, w_gate_up, w_down, retain_gate, retain_up, retain_down,
           forget_gate, forget_up, forget_down, scales, token_experiment_ids):
    return _run(x, w_gate_up, w_down, retain_gate, retain_up, retain_down,
                forget_gate, forget_up, forget_down, scales,
                token_experiment_ids)
